# baseline (speedup 1.0000x reference)
import jax
import jax.numpy as jnp
from jax import lax
from jax.experimental import pallas as pl
from jax.experimental.pallas import tpu as pltpu
from jax.experimental.pallas import tpu_sc as plsc

_B, _C_IN, _H, _W = 32, 96, 56, 56
_C_OUT = 128
_L = 16
_NCHUNK = _C_OUT // _L


def _sc_compact_body(w_hbm, s_hbm, w_v, s_v):
    cid = lax.axis_index("c")
    sid = lax.axis_index("s")

    @pl.when((cid == 0) & (sid == 0))
    def _():
        pltpu.sync_copy(w_hbm, w_v)
        lanes = lax.iota(jnp.int32, _L)
        fifteen = jnp.minimum(lanes + _L, _L - 1)

        w_list, cs_list, cum_list = [], [], []
        carry = jnp.zeros((_L,), jnp.float32)
        for j in range(_NCHUNK):
            w = w_v[pl.ds(j * _L, _L)]
            x = jnp.where(w != 0.0, 1.0, 0.0)
            for k in (1, 2, 4, 8):
                idx = jnp.maximum(lanes - k, 0)
                x = x + jnp.where(lanes >= k, x[idx], 0.0)
            x = x + carry
            carry = x[fifteen]
            w_list.append(w)
            cs_list.append(x)
            cum_list.append(carry)

        for t in range(_NCHUNK):
            targets = (lanes + t * _L + 1).astype(jnp.float32)
            s = jnp.zeros((_L,), jnp.float32)
            prev_cum = jnp.zeros((_L,), jnp.float32)
            for j in range(_NCHUNK):
                csj = cs_list[j]
                pos = jnp.zeros((_L,), jnp.int32)
                for step in (8, 4, 2, 1):
                    mid = jnp.minimum(pos + (step - 1), _L - 1)
                    probe = csj[mid]
                    pos = jnp.where(probe < targets, pos + step, pos)
                val = w_list[j][jnp.minimum(pos, _L - 1)]
                pick = (prev_cum < targets) & (targets <= cum_list[j])
                s = jnp.where(pick, val, s)
                prev_cum = cum_list[j]
            s_v[pl.ds(t * _L, _L)] = s

        pltpu.sync_copy(s_v, s_hbm)


def _sc_compact(weight):
    mesh = plsc.VectorSubcoreMesh(core_axis_name="c", subcore_axis_name="s")
    return pl.kernel(
        _sc_compact_body,
        jax.ShapeDtypeStruct((_C_OUT,), jnp.float32),
        mesh=mesh,
        scratch_types=[
            pltpu.VMEM((_C_OUT,), jnp.float32),
            pltpu.VMEM((_C_OUT,), jnp.float32),
        ],
    )(weight)


def _tc_body(s_ref, in_ref, out_ref):
    scale = s_ref[:, 0:_C_IN].reshape(1, 1, 1, _C_IN)
    out_ref[:, :, :, 0:_C_IN] = in_ref[:, :, :, :] * scale
    out_ref[:, :, :, _C_IN:_C_OUT] = jnp.zeros(
        (8, _H, _W, _C_OUT - _C_IN), dtype=out_ref.dtype
    )


def kernel(input, weight_kse, weight):
    del weight_kse
    s_row = _sc_compact(weight).reshape(1, _C_OUT)
    xt = jnp.transpose(input, (0, 2, 3, 1))

    out_t = pl.pallas_call(
        _tc_body,
        grid=(_B // 8,),
        in_specs=[
            pl.BlockSpec((1, _C_OUT), lambda b: (0, 0)),
            pl.BlockSpec((8, _H, _W, _C_IN), lambda b: (b, 0, 0, 0)),
        ],
        out_specs=pl.BlockSpec((8, _H, _W, _C_OUT), lambda b: (b, 0, 0, 0)),
        out_shape=jax.ShapeDtypeStruct((_B, _H, _W, _C_OUT), input.dtype),
        compiler_params=pltpu.CompilerParams(
            dimension_semantics=("arbitrary",),
        ),
    )(s_row, xt)
    return jnp.transpose(out_t, (0, 3, 1, 2))


# speedup vs baseline: 1.6072x; 1.6072x over previous
import jax
import jax.numpy as jnp
from jax.experimental import pallas as pl
from jax.experimental.pallas import tpu as pltpu

_B, _C_IN, _H, _W = 32, 96, 56, 56
_C_OUT = 128


def _compact_scale_row(w_row, w_col):
    lane = jax.lax.broadcasted_iota(jnp.int32, (_C_OUT, _C_OUT), 1)
    sub = jax.lax.broadcasted_iota(jnp.int32, (_C_OUT, _C_OUT), 0)
    incl = jnp.where((lane <= sub) & (w_row != 0.0), 1.0, 0.0)
    cs_col = jnp.sum(incl, axis=1, keepdims=True)
    lanef = (lane + 1).astype(jnp.float32)
    pick = (cs_col == lanef) & (w_col != 0.0)
    return jnp.sum(jnp.where(pick, w_col, 0.0), axis=0, keepdims=True)


def _body(w_row_ref, w_col_ref, in_ref, out_ref, s_ref):
    @pl.when(pl.program_id(0) == 0)
    def _():
        s_ref[:, :] = _compact_scale_row(w_row_ref[:, :], w_col_ref[:, :])

    scale = s_ref[:, 0:_C_IN].reshape(1, 1, _C_IN)
    out_ref[:, :, :, 0:_C_IN] = in_ref[:, :, :, :] * scale.reshape(1, 1, 1, _C_IN)
    out_ref[:, :, :, _C_IN:_C_OUT] = jnp.zeros(
        (8, _H, _W, _C_OUT - _C_IN), dtype=out_ref.dtype
    )


def kernel(input, weight_kse, weight):
    del weight_kse
    w_row = weight.reshape(1, _C_OUT)
    w_col = weight.reshape(_C_OUT, 1)
    xt = jnp.transpose(input, (0, 2, 3, 1))

    out_t = pl.pallas_call(
        _body,
        grid=(_B // 8,),
        in_specs=[
            pl.BlockSpec((1, _C_OUT), lambda b: (0, 0)),
            pl.BlockSpec((_C_OUT, 1), lambda b: (0, 0)),
            pl.BlockSpec((8, _H, _W, _C_IN), lambda b: (b, 0, 0, 0)),
        ],
        out_specs=pl.BlockSpec((8, _H, _W, _C_OUT), lambda b: (b, 0, 0, 0)),
        out_shape=jax.ShapeDtypeStruct((_B, _H, _W, _C_OUT), input.dtype),
        scratch_shapes=[pltpu.VMEM((1, _C_OUT), jnp.float32)],
        compiler_params=pltpu.CompilerParams(
            dimension_semantics=("arbitrary",),
        ),
    )(w_row, w_col, xt)
    return jnp.transpose(out_t, (0, 3, 1, 2))
